# baseline (speedup 1.0000x reference)
import functools

import jax
import jax.numpy as jnp
from jax import lax
from jax.experimental import pallas as pl
from jax.experimental.pallas import tpu as pltpu
from jax.experimental.pallas import tpu_sc as plsc

_EPS = 1e-05


def _stats_body(x_ref, mean_ref, s_ref):
    xv = x_ref[0]
    c = xv.shape[0]
    mean = jnp.mean(xv, axis=0, keepdims=True)
    d = xv - mean
    var = jnp.sum(d * d, axis=0, keepdims=True) * (1.0 / (c - 1))
    mean_ref[0] = mean
    s_ref[0] = jnp.sqrt(var + _EPS)


def _affine_body(idx_ref, mean_n, mean_j, s_n, s_j, scale_ref, bias_ref,
                 ridx_ref):
    sc = s_n[...] / s_j[...]
    scale_ref[...] = sc
    bias_ref[...] = mean_n[...] - mean_j[...] * sc
    j = idx_ref[pl.program_id(0)]
    ridx_ref[...] = (j * 128
                     + 8 * lax.broadcasted_iota(jnp.int32, (16, 8), 0)
                     + lax.broadcasted_iota(jnp.int32, (16, 8), 1))


def _sc_main(x_hbm, scale_hbm, bias_hbm, ridx_hbm, out_hbm,
             idx_v, sv, bv, buf0, buf1, g0, g1, o0, o1):
    wid = lax.axis_index("s") * 2 + lax.axis_index("c")
    bufs = (buf0, buf1)
    gsems = (g0, g1)
    osems = (o0, o1)
    for t in range(2):
        n = wid * 2 + t
        pltpu.sync_copy(ridx_hbm.at[pl.ds(n * 16, 16)], idx_v)
        pltpu.sync_copy(scale_hbm.at[pl.ds(n * 4096, 4096)], sv)
        pltpu.sync_copy(bias_hbm.at[pl.ds(n * 4096, 4096)], bv)
        g = [None, None]
        o = [None, None]
        g[0] = pltpu.async_copy(x_hbm.at[idx_v.at[0]], buf0, g0)
        for k in range(16):
            cur = k & 1
            nxt = (k + 1) & 1
            g[cur].wait()
            if k + 1 < 16:
                if o[nxt] is not None:
                    o[nxt].wait()
                    o[nxt] = None
                g[nxt] = pltpu.async_copy(
                    x_hbm.at[idx_v.at[k + 1]], bufs[nxt], gsems[nxt])
            buf = bufs[cur]

            def body(i, carry):
                s16 = sv[pl.ds(i * 16, 16)]
                b16 = bv[pl.ds(i * 16, 16)]
                for r in range(8):
                    buf[r, pl.ds(i * 16, 16)] = (
                        buf[r, pl.ds(i * 16, 16)] * s16 + b16)
                return carry

            lax.fori_loop(0, 256, body, 0)
            o[cur] = pltpu.async_copy(
                buf, out_hbm.at[pl.ds(n * 128 + k * 8, 8)], osems[cur])
        for b in range(2):
            if o[b] is not None:
                o[b].wait()


def kernel(x, idx_swap):
    n, c, h, w = x.shape
    hw = h * w
    x3 = x.reshape(n, c, hw)

    mean, s = pl.pallas_call(
        _stats_body,
        grid=(n,),
        in_specs=[pl.BlockSpec((1, c, hw), lambda i: (i, 0, 0))],
        out_specs=[
            pl.BlockSpec((1, 1, hw), lambda i: (i, 0, 0)),
            pl.BlockSpec((1, 1, hw), lambda i: (i, 0, 0)),
        ],
        out_shape=[
            jax.ShapeDtypeStruct((n, 1, hw), jnp.float32),
            jax.ShapeDtypeStruct((n, 1, hw), jnp.float32),
        ],
    )(x3)

    grid_spec = pltpu.PrefetchScalarGridSpec(
        num_scalar_prefetch=1,
        grid=(n,),
        in_specs=[
            pl.BlockSpec((1, 1, hw), lambda i, idx_ref: (i, 0, 0)),
            pl.BlockSpec((1, 1, hw), lambda i, idx_ref: (idx_ref[i], 0, 0)),
            pl.BlockSpec((1, 1, hw), lambda i, idx_ref: (i, 0, 0)),
            pl.BlockSpec((1, 1, hw), lambda i, idx_ref: (idx_ref[i], 0, 0)),
        ],
        out_specs=[
            pl.BlockSpec((1, 1, hw), lambda i, idx_ref: (i, 0, 0)),
            pl.BlockSpec((1, 1, hw), lambda i, idx_ref: (i, 0, 0)),
            pl.BlockSpec((16, 8), lambda i, idx_ref: (i, 0)),
        ],
    )
    scale, bias, ridx = pl.pallas_call(
        _affine_body,
        grid_spec=grid_spec,
        out_shape=[
            jax.ShapeDtypeStruct((n, 1, hw), jnp.float32),
            jax.ShapeDtypeStruct((n, 1, hw), jnp.float32),
            jax.ShapeDtypeStruct((n * 16, 8), jnp.int32),
        ],
    )(idx_swap, mean, mean, s, s)

    x2 = x3.reshape(n * c, hw)
    mesh = plsc.VectorSubcoreMesh(core_axis_name="c", subcore_axis_name="s")
    sc_call = functools.partial(
        pl.kernel,
        mesh=mesh,
        out_type=jax.ShapeDtypeStruct((n * c, hw), jnp.float32),
        scratch_types=[
            pltpu.VMEM((16, 8), jnp.int32),
            pltpu.VMEM((hw,), jnp.float32),
            pltpu.VMEM((hw,), jnp.float32),
            pltpu.VMEM((8, hw), jnp.float32),
            pltpu.VMEM((8, hw), jnp.float32),
            pltpu.SemaphoreType.DMA,
            pltpu.SemaphoreType.DMA,
            pltpu.SemaphoreType.DMA,
            pltpu.SemaphoreType.DMA,
        ],
    )(_sc_main)
    out2 = sc_call(x2, scale.reshape(-1), bias.reshape(-1), ridx)
    return out2.reshape(n, c, h, w)


# speedup vs baseline: 2.8253x; 2.8253x over previous
import jax
import jax.numpy as jnp
from jax import lax
from jax.experimental import pallas as pl
from jax.experimental.pallas import tpu as pltpu

_EPS = 1e-05
_K = 4


def _body(idx_ref, x_hbm, out_hbm, inb, outb, mean_ref, s_ref,
          sem_in, sem_out):
    nb = x_hbm.shape[0]
    c = x_hbm.shape[1]

    def in_copy(n, sl):
        return pltpu.make_async_copy(
            x_hbm.at[pl.ds(n, 1)], inb.at[pl.ds(sl, 1)], sem_in.at[sl])

    def out_copy(m, sl):
        return pltpu.make_async_copy(
            outb.at[pl.ds(sl, 1)], out_hbm.at[pl.ds(m, 1)], sem_out.at[sl])

    def prime1(n, _):
        in_copy(n, n).start()
        return 0

    lax.fori_loop(0, _K, prime1, 0)

    def phase1(n, _):
        sl = lax.rem(n, _K)
        in_copy(n, sl).wait()
        xv = inb[sl]
        mean = jnp.mean(xv, axis=0, keepdims=True)
        d = xv - mean
        var = jnp.sum(d * d, axis=0, keepdims=True) * (1.0 / (c - 1))
        mean_ref[pl.ds(n, 1)] = mean
        s_ref[pl.ds(n, 1)] = jnp.sqrt(var + _EPS)

        @pl.when(n + _K < nb)
        def _():
            in_copy(n + _K, sl).start()

        return 0

    lax.fori_loop(0, nb, phase1, 0)

    def prime2(m, _):
        in_copy(idx_ref[m], m).start()
        return 0

    lax.fori_loop(0, _K, prime2, 0)

    def phase2(m, _):
        sl = lax.rem(m, _K)
        j = idx_ref[m]
        in_copy(j, sl).wait()

        @pl.when(m >= _K)
        def _():
            out_copy(m - _K, sl).wait()

        xg = inb[sl]
        mean_j = mean_ref[pl.ds(j, 1)]
        mean_m = mean_ref[pl.ds(m, 1)]
        scale = s_ref[pl.ds(m, 1)] / s_ref[pl.ds(j, 1)]
        outb[sl] = (xg - mean_j) * scale + mean_m
        out_copy(m, sl).start()

        @pl.when(m + _K < nb)
        def _():
            in_copy(idx_ref[m + _K], sl).start()

        return 0

    lax.fori_loop(0, nb, phase2, 0)

    def drain(k, _):
        m = nb - _K + k
        out_copy(m, lax.rem(m, _K)).wait()
        return 0

    lax.fori_loop(0, _K, drain, 0)


def kernel(x, idx_swap):
    n, c, h, w = x.shape
    hw = h * w
    x3 = x.reshape(n, c, hw)
    grid_spec = pltpu.PrefetchScalarGridSpec(
        num_scalar_prefetch=1,
        grid=(1,),
        in_specs=[pl.BlockSpec(memory_space=pl.ANY)],
        out_specs=pl.BlockSpec(memory_space=pl.ANY),
        scratch_shapes=[
            pltpu.VMEM((_K, c, hw), jnp.float32),
            pltpu.VMEM((_K, c, hw), jnp.float32),
            pltpu.VMEM((n, hw), jnp.float32),
            pltpu.VMEM((n, hw), jnp.float32),
            pltpu.SemaphoreType.DMA((_K,)),
            pltpu.SemaphoreType.DMA((_K,)),
        ],
    )
    out = pl.pallas_call(
        _body,
        grid_spec=grid_spec,
        out_shape=jax.ShapeDtypeStruct((n, c, hw), x.dtype),
    )(idx_swap, x3)
    return out.reshape(n, c, h, w)
